# baseline (speedup 1.0000x reference)
import functools

import jax
import jax.numpy as jnp
from jax import lax
from jax.experimental import pallas as pl
from jax.experimental.pallas import tpu as pltpu
from jax.experimental.pallas import tpu_sc as plsc


def _make_sc_kernel(T, D, n_workers, num_cores, chunk):
    t_per_w = T // n_workers
    n_chunks = t_per_w // chunk
    mesh = plsc.VectorSubcoreMesh(core_axis_name="c", subcore_axis_name="s")

    @functools.partial(
        pl.kernel,
        mesh=mesh,
        compiler_params=pltpu.CompilerParams(use_tc_tiling_on_sc=False),
        out_type=jax.ShapeDtypeStruct((T, D), jnp.float32),
        scratch_types=[
            pltpu.VMEM((chunk,), jnp.int32),
            pltpu.VMEM((chunk, D), jnp.float32),
            pltpu.VMEM((chunk, D), jnp.float32),
            pltpu.VMEM((chunk, D), jnp.float32),
            pltpu.VMEM((chunk, D), jnp.float32),
            pltpu.SemaphoreType.DMA,
        ],
    )
    def sc_kernel(idx_hbm, w_hbm, s_hbm, eps_hbm, out_hbm,
                  idx_v, mu_v, lv_v, eps_v, o_v, sem):
        wid = lax.axis_index("s") * num_cores + lax.axis_index("c")
        base_w = wid * t_per_w

        def chunk_body(ci, carry):
            base = base_w + ci * chunk
            pltpu.sync_copy(idx_hbm.at[pl.ds(base, chunk)], idx_v)
            g_mu = pltpu.async_copy(w_hbm.at[idx_v], mu_v, sem)
            g_lv = pltpu.async_copy(s_hbm.at[idx_v], lv_v, sem)
            pltpu.sync_copy(eps_hbm.at[pl.ds(base, chunk)], eps_v)
            g_mu.wait()
            g_lv.wait()

            def row_body(r, c2):
                for j in range(D // 16):
                    sl = pl.ds(j * 16, 16)
                    std = jnp.exp(0.5 * lv_v[r, sl])
                    o_v[r, sl] = eps_v[r, sl] * std + mu_v[r, sl]
                return c2

            lax.fori_loop(0, chunk, row_body, 0, unroll=2)
            pltpu.sync_copy(o_v, out_hbm.at[pl.ds(base, chunk)])
            return carry

        lax.fori_loop(0, n_chunks, chunk_body, 0)

    return sc_kernel


def kernel(indices, weight, spread, eps):
    B, H = indices.shape
    V, D = weight.shape
    T = B * H

    idx_flat = indices.reshape(T)
    eps_flat = eps.reshape(T, D)

    info = plsc.get_sparse_core_info()
    n_workers = info.num_cores * info.num_subcores
    chunk = 128

    sc_kernel = _make_sc_kernel(T, D, n_workers, info.num_cores, chunk)
    out_flat = sc_kernel(idx_flat, weight, spread, eps_flat)
    return out_flat.reshape(B, H, D)


# speedup vs baseline: 1.1390x; 1.1390x over previous
import functools

import jax
import jax.numpy as jnp
from jax import lax
from jax.experimental import pallas as pl
from jax.experimental.pallas import tpu as pltpu
from jax.experimental.pallas import tpu_sc as plsc

_CHUNK = 128


def _make_sc_kernel(T, D, n_workers, num_cores):
    chunk = _CHUNK
    t_per_w = T // n_workers
    n = t_per_w // chunk
    mesh = plsc.VectorSubcoreMesh(core_axis_name="c", subcore_axis_name="s")

    @functools.partial(
        pl.kernel,
        mesh=mesh,
        compiler_params=pltpu.CompilerParams(use_tc_tiling_on_sc=False),
        out_type=jax.ShapeDtypeStruct((T * D,), jnp.float32),
        scratch_types=[
            pltpu.VMEM((chunk,), jnp.int32),
            pltpu.VMEM((chunk,), jnp.int32),
            pltpu.VMEM((chunk, D), jnp.float32),
            pltpu.VMEM((chunk, D), jnp.float32),
            pltpu.VMEM((chunk, D), jnp.float32),
            pltpu.VMEM((chunk, D), jnp.float32),
            pltpu.VMEM((chunk * D,), jnp.float32),
            pltpu.VMEM((chunk * D,), jnp.float32),
            pltpu.VMEM((chunk * D,), jnp.float32),
            pltpu.VMEM((chunk * D,), jnp.float32),
            pltpu.SemaphoreType.DMA,
            pltpu.SemaphoreType.DMA,
            pltpu.SemaphoreType.DMA,
            pltpu.SemaphoreType.DMA,
            pltpu.SemaphoreType.DMA,
        ],
    )
    def sc_kernel(idx_hbm, w_hbm, s_hbm, eps_hbm, out_hbm,
                  idx0, idx1, mu0, mu1, lv0, lv1, eps0, eps1, o0, o1,
                  sem_idx, sem_mu, sem_lv, sem_eps, sem_out):
        wid = lax.axis_index("s") * num_cores + lax.axis_index("c")
        base_w = wid * t_per_w

        idx_b = (idx0, idx1)
        mu_b = (mu0, mu1)
        lv_b = (lv0, lv1)
        eps_b = (eps0, eps1)
        o_b = (o0, o1)

        def idx_start(ci, s):
            src = idx_hbm.at[pl.ds(base_w + ci * chunk, chunk)]
            pltpu.make_async_copy(src, idx_b[s], sem_idx).start()

        def idx_wait(s):
            src = idx_hbm.at[pl.ds(0, chunk)]
            pltpu.make_async_copy(src, idx_b[s], sem_idx).wait()

        def gather_start(ci, s):
            pltpu.make_async_copy(w_hbm.at[idx_b[s]], mu_b[s], sem_mu).start()
            pltpu.make_async_copy(s_hbm.at[idx_b[s]], lv_b[s], sem_lv).start()
            src = eps_hbm.at[pl.ds((base_w + ci * chunk) * D, chunk * D)]
            pltpu.make_async_copy(src, eps_b[s], sem_eps).start()

        def gather_wait(s):
            pltpu.make_async_copy(w_hbm.at[idx_b[s]], mu_b[s], sem_mu).wait()
            pltpu.make_async_copy(s_hbm.at[idx_b[s]], lv_b[s], sem_lv).wait()
            src = eps_hbm.at[pl.ds(0, chunk * D)]
            pltpu.make_async_copy(src, eps_b[s], sem_eps).wait()

        def out_start(ci, s):
            dst = out_hbm.at[pl.ds((base_w + ci * chunk) * D, chunk * D)]
            pltpu.make_async_copy(o_b[s], dst, sem_out).start()

        def out_wait(s):
            dst = out_hbm.at[pl.ds(0, chunk * D)]
            pltpu.make_async_copy(o_b[s], dst, sem_out).wait()

        def compute(s):
            mu_v, lv_v, eps_v, o_v = mu_b[s], lv_b[s], eps_b[s], o_b[s]

            def row_body(r, carry):
                rb = r * D
                for j in range(D // 16):
                    sl = pl.ds(j * 16, 16)
                    fl = pl.ds(rb + j * 16, 16)
                    std = jnp.exp(0.5 * lv_v[r, sl])
                    o_v[fl] = eps_v[fl] * std + mu_v[r, sl]
                return carry

            lax.fori_loop(0, chunk, row_body, 0, unroll=2)

        idx_start(0, 0)
        idx_wait(0)
        gather_start(0, 0)
        idx_start(1, 1)

        idx_wait(1)
        gather_start(1, 1)
        gather_wait(0)
        idx_start(2, 0)
        compute(0)
        out_start(0, 0)

        idx_wait(0)
        gather_start(2, 0)
        gather_wait(1)
        idx_start(3, 1)
        compute(1)
        out_start(1, 1)

        def pair_body(p, carry):
            for k in range(2):
                ci = 2 + 2 * p + k
                idx_wait(1 - k)
                gather_start(ci + 1, 1 - k)
                gather_wait(k)
                idx_start(ci + 2, k)
                out_wait(k)
                compute(k)
                out_start(ci, k)
            return carry

        lax.fori_loop(0, (n - 4) // 2, pair_body, 0)

        idx_wait(1)
        gather_start(n - 1, 1)
        gather_wait(0)
        out_wait(0)
        compute(0)
        out_start(n - 2, 0)

        gather_wait(1)
        out_wait(1)
        compute(1)
        out_start(n - 1, 1)

        out_wait(0)
        out_wait(1)

    return sc_kernel


def kernel(indices, weight, spread, eps):
    B, H = indices.shape
    V, D = weight.shape
    T = B * H

    idx_flat = indices.reshape(T)
    eps_flat = eps.reshape(T * D)

    info = plsc.get_sparse_core_info()
    n_workers = info.num_cores * info.num_subcores

    sc_kernel = _make_sc_kernel(T, D, n_workers, info.num_cores)
    out_flat = sc_kernel(idx_flat, weight, spread, eps_flat)
    return out_flat.reshape(B, H, D)


# speedup vs baseline: 1.6981x; 1.4909x over previous
import functools

import jax
import jax.numpy as jnp
from jax import lax
from jax.experimental import pallas as pl
from jax.experimental.pallas import tpu as pltpu
from jax.experimental.pallas import tpu_sc as plsc

_CHUNK = 128


def _make_sc_kernel(T, D, n_workers, num_cores):
    chunk = _CHUNK
    t_per_w = T // n_workers
    n = t_per_w // chunk
    mesh = plsc.VectorSubcoreMesh(core_axis_name="c", subcore_axis_name="s")

    @functools.partial(
        pl.kernel,
        mesh=mesh,
        compiler_params=pltpu.CompilerParams(use_tc_tiling_on_sc=False),
        out_type=jax.ShapeDtypeStruct((T * D,), jnp.float32),
        scratch_types=[
            pltpu.VMEM((chunk,), jnp.int32),
            pltpu.VMEM((chunk,), jnp.int32),
            pltpu.VMEM((chunk, D), jnp.float32),
            pltpu.VMEM((chunk, D), jnp.float32),
            pltpu.VMEM((chunk, D), jnp.float32),
            pltpu.VMEM((chunk, D), jnp.float32),
            pltpu.VMEM((chunk * D,), jnp.float32),
            pltpu.VMEM((chunk * D,), jnp.float32),
            pltpu.VMEM((chunk * D,), jnp.float32),
            pltpu.VMEM((chunk * D,), jnp.float32),
            pltpu.SemaphoreType.DMA,
            pltpu.SemaphoreType.DMA,
            pltpu.SemaphoreType.DMA,
            pltpu.SemaphoreType.DMA,
            pltpu.SemaphoreType.DMA,
        ],
    )
    def sc_kernel(idx_hbm, w_hbm, s_hbm, eps_hbm, out_hbm,
                  idx0, idx1, mu0, mu1, lv0, lv1, eps0, eps1, o0, o1,
                  sem_idx, sem_mu, sem_lv, sem_eps, sem_out):
        wid = lax.axis_index("s") * num_cores + lax.axis_index("c")
        base_w = wid * t_per_w

        idx_b = (idx0, idx1)
        mu_b = (mu0, mu1)
        lv_b = (lv0, lv1)
        eps_b = (eps0, eps1)
        o_b = (o0, o1)

        def idx_start(ci, s):
            src = idx_hbm.at[pl.ds(base_w + ci * chunk, chunk)]
            pltpu.make_async_copy(src, idx_b[s], sem_idx).start()

        def idx_wait(s):
            src = idx_hbm.at[pl.ds(0, chunk)]
            pltpu.make_async_copy(src, idx_b[s], sem_idx).wait()

        def gather_start(ci, s):
            pltpu.make_async_copy(w_hbm.at[idx_b[s]], mu_b[s], sem_mu).start()
            pltpu.make_async_copy(s_hbm.at[idx_b[s]], lv_b[s], sem_lv).start()
            src = eps_hbm.at[pl.ds((base_w + ci * chunk) * D, chunk * D)]
            pltpu.make_async_copy(src, eps_b[s], sem_eps).start()

        def gather_wait(s):
            pltpu.make_async_copy(w_hbm.at[idx_b[s]], mu_b[s], sem_mu).wait()
            pltpu.make_async_copy(s_hbm.at[idx_b[s]], lv_b[s], sem_lv).wait()
            src = eps_hbm.at[pl.ds(0, chunk * D)]
            pltpu.make_async_copy(src, eps_b[s], sem_eps).wait()

        def out_start(ci, s):
            dst = out_hbm.at[pl.ds((base_w + ci * chunk) * D, chunk * D)]
            pltpu.make_async_copy(o_b[s], dst, sem_out).start()

        def out_wait(s):
            dst = out_hbm.at[pl.ds(0, chunk * D)]
            pltpu.make_async_copy(o_b[s], dst, sem_out).wait()

        def compute(s):
            mu_v, lv_v, eps_v, o_v = mu_b[s], lv_b[s], eps_b[s], o_b[s]

            @plsc.parallel_loop(0, chunk, 1, unroll=4)
            def row_body(r):
                rb = r * D
                for j in range(D // 16):
                    sl = pl.ds(j * 16, 16)
                    fl = pl.ds(rb + j * 16, 16)
                    std = jnp.exp(0.5 * lv_v[r, sl])
                    o_v[fl] = eps_v[fl] * std + mu_v[r, sl]

        idx_start(0, 0)
        idx_wait(0)
        gather_start(0, 0)
        idx_start(1, 1)

        idx_wait(1)
        gather_start(1, 1)
        gather_wait(0)
        idx_start(2, 0)
        compute(0)
        out_start(0, 0)

        idx_wait(0)
        gather_start(2, 0)
        gather_wait(1)
        idx_start(3, 1)
        compute(1)
        out_start(1, 1)

        def pair_body(p, carry):
            for k in range(2):
                ci = 2 + 2 * p + k
                idx_wait(1 - k)
                gather_start(ci + 1, 1 - k)
                gather_wait(k)
                idx_start(ci + 2, k)
                out_wait(k)
                compute(k)
                out_start(ci, k)
            return carry

        lax.fori_loop(0, (n - 4) // 2, pair_body, 0)

        idx_wait(1)
        gather_start(n - 1, 1)
        gather_wait(0)
        out_wait(0)
        compute(0)
        out_start(n - 2, 0)

        gather_wait(1)
        out_wait(1)
        compute(1)
        out_start(n - 1, 1)

        out_wait(0)
        out_wait(1)

    return sc_kernel


def kernel(indices, weight, spread, eps):
    B, H = indices.shape
    V, D = weight.shape
    T = B * H

    idx_flat = indices.reshape(T)
    eps_flat = eps.reshape(T * D)

    info = plsc.get_sparse_core_info()
    n_workers = info.num_cores * info.num_subcores

    sc_kernel = _make_sc_kernel(T, D, n_workers, info.num_cores)
    out_flat = sc_kernel(idx_flat, weight, spread, eps_flat)
    return out_flat.reshape(B, H, D)
